# baseline (speedup 1.0000x reference)
import functools

import jax
import jax.numpy as jnp
from jax import lax
from jax.experimental import pallas as pl
from jax.experimental.pallas import tpu as pltpu
from jax.experimental.pallas import tpu_sc as plsc

VOCAB = 100000
EMB = 64
NUM_CLASS = 2
B = 16384
L = 200

NC = 2
NS = 16
NW = NC * NS
BAGS_PER_TEC = B // NW
GROUP = 16
CHUNK = 128
NCHUNK = BAGS_PER_TEC // CHUNK
SUBG = CHUNK // GROUP
RSLICE = 40
NSLICE = L // RSLICE
TSLICES = NCHUNK * NSLICE
NBUF = 4
POS_UNROLL = 4
PBLK = 20480
VOCAB_PAD = 102400
PGRID = VOCAB_PAD // PBLK


def _rne_bf16_hi(x_f32):
    u = lax.bitcast_convert_type(x_f32, jnp.int32)
    lsb = lax.shift_right_logical(u, 16) & 1
    r = u + jnp.int32(0x7FFF) + lsb
    return r & jnp.int32(-65536)


def _proj_body(embt_ref, w_ref, b_ref, out_ref, bias_ref):
    pf = lax.dot_general(
        w_ref[...], embt_ref[...], (((1,), (0,)), ((), ())),
        preferred_element_type=jnp.float32)
    hi0 = _rne_bf16_hi(pf[0:1, :])
    hi1 = _rne_bf16_hi(pf[1:2, :])
    packed = hi1 | lax.shift_right_logical(hi0, 16)
    out_ref[...] = packed.reshape(PBLK)

    @pl.when(pl.program_id(0) == 0)
    def _():
        bias_ref[...] = jnp.broadcast_to(b_ref[...], (NUM_CLASS, 16))


def _project(emb_table_t, fc_w, fc_b2):
    return pl.pallas_call(
        _proj_body,
        grid=(PGRID,),
        in_specs=[
            pl.BlockSpec((EMB, PBLK), lambda i: (0, i)),
            pl.BlockSpec((NUM_CLASS, EMB), lambda i: (0, 0)),
            pl.BlockSpec((NUM_CLASS, 1), lambda i: (0, 0)),
        ],
        out_specs=(
            pl.BlockSpec((PBLK,), lambda i: (i,)),
            pl.BlockSpec((NUM_CLASS, 16), lambda i: (0, 0)),
        ),
        out_shape=(
            jax.ShapeDtypeStruct((VOCAB_PAD,), jnp.int32),
            jax.ShapeDtypeStruct((NUM_CLASS, 16), jnp.float32),
        ),
    )(emb_table_t, fc_w, fc_b2)


def _bags_body(ptab_hbm, tokt_hbm, bias_hbm, out_hbm,
               tab_sh, table_v, buf0_v, buf1_v, buf2_v, buf3_v,
               res_v, acc_v, bias_v, sem0, sem1, sem2, sem3):
    c = lax.axis_index("c")
    s = lax.axis_index("s")
    wid = s * NC + c
    col0 = wid * BAGS_PER_TEC
    sems = (sem0, sem1, sem2, sem3)
    bufs = (buf0_v, buf1_v, buf2_v, buf3_v)

    def start_slice_copy(t, bb):
        col = col0 + (t // NSLICE) * CHUNK
        r0 = (t % NSLICE) * RSLICE
        pltpu.async_copy(
            tokt_hbm.at[pl.ds(r0, RSLICE), pl.ds(col, CHUNK)],
            bufs[bb], sems[bb])

    def wait_slice_copy(bb):
        pltpu.make_async_copy(
            tokt_hbm.at[pl.ds(0, RSLICE), pl.ds(0, CHUNK)],
            bufs[bb], sems[bb]).wait()

    for bb in range(NBUF):
        start_slice_copy(bb, bb)

    @pl.when(s == 0)
    def _():
        pltpu.sync_copy(ptab_hbm, tab_sh)

    plsc.subcore_barrier()
    pltpu.sync_copy(tab_sh, table_v)
    pltpu.sync_copy(bias_hbm, bias_v)

    iota16 = lax.iota(jnp.int32, 16)
    b0 = bias_v[0, :]
    b1 = bias_v[1, :]
    inv_l = jnp.float32(1.0 / L)
    zerov = jnp.zeros((16,), jnp.float32)

    for j in range(SUBG):
        acc_v[pl.ds(j * GROUP, GROUP)] = zerov
        acc_v[pl.ds(CHUNK + j * GROUP, GROUP)] = zerov

    @pl.loop(0, TSLICES, step=NBUF)
    def _outer(t4):
        for bb in range(NBUF):
            t = t4 + bb
            wait_slice_copy(bb)
            rc = lax.rem(t, NSLICE)
            g = lax.div(t, NSLICE)

            @pl.loop(0, SUBG)
            def _groups(j, bb=bb):
                a0 = acc_v[pl.ds(j * GROUP, GROUP)]
                a1 = acc_v[pl.ds(CHUNK + j * GROUP, GROUP)]

                def pos_body(i, carry, bb=bb, j=j):
                    a0, a1, c0, c1 = carry
                    for k in range(POS_UNROLL):
                        p = i * POS_UNROLL + k
                        toks = bufs[bb][p, pl.ds(j * GROUP, GROUP)]
                        packed = plsc.load_gather(table_v, [toks])
                        lo = plsc.bitcast(packed << 16, jnp.float32)
                        hi = plsc.bitcast(packed & jnp.int32(-65536),
                                          jnp.float32)
                        if k % 2 == 0:
                            a0 = a0 + lo
                            a1 = a1 + hi
                        else:
                            c0 = c0 + lo
                            c1 = c1 + hi
                    return a0, a1, c0, c1

                a0, a1, c0, c1 = lax.fori_loop(
                    0, RSLICE // POS_UNROLL, pos_body,
                    (a0, a1, zerov, zerov))
                acc_v[pl.ds(j * GROUP, GROUP)] = a0 + c0
                acc_v[pl.ds(CHUNK + j * GROUP, GROUP)] = a1 + c1

            tt = t + NBUF

            @pl.when(tt < TSLICES)
            def _(bb=bb, tt=tt):
                start_slice_copy(tt, bb)

            @pl.when(rc == NSLICE - 1)
            def _(g=g):
                @pl.loop(0, SUBG)
                def _fin(j):
                    a0 = acc_v[pl.ds(j * GROUP, GROUP)]
                    a1 = acc_v[pl.ds(CHUNK + j * GROUP, GROUP)]
                    base = g * CHUNK + j * GROUP
                    res_v[0, pl.ds(base, GROUP)] = a0 * inv_l + b0
                    res_v[1, pl.ds(base, GROUP)] = a1 * inv_l + b1
                    acc_v[pl.ds(j * GROUP, GROUP)] = zerov
                    acc_v[pl.ds(CHUNK + j * GROUP, GROUP)] = zerov

    pltpu.sync_copy(res_v, out_hbm.at[:, pl.ds(col0, BAGS_PER_TEC)])


_bags = functools.partial(
    pl.kernel,
    out_type=jax.ShapeDtypeStruct((NUM_CLASS, B), jnp.float32),
    mesh=plsc.VectorSubcoreMesh(
        core_axis_name="c", subcore_axis_name="s",
        num_cores=NC, num_subcores=NS),
    compiler_params=pltpu.CompilerParams(needs_layout_passes=False),
    scratch_types=[
        pltpu.VMEM_SHARED((VOCAB_PAD,), jnp.int32),
        pltpu.VMEM((VOCAB_PAD,), jnp.int32),
        pltpu.VMEM((RSLICE, CHUNK), jnp.int32),
        pltpu.VMEM((RSLICE, CHUNK), jnp.int32),
        pltpu.VMEM((RSLICE, CHUNK), jnp.int32),
        pltpu.VMEM((RSLICE, CHUNK), jnp.int32),
        pltpu.VMEM((NUM_CLASS, BAGS_PER_TEC), jnp.float32),
        pltpu.VMEM((CHUNK * NUM_CLASS,), jnp.float32),
        pltpu.VMEM((NUM_CLASS, 16), jnp.float32),
        pltpu.SemaphoreType.DMA,
        pltpu.SemaphoreType.DMA,
        pltpu.SemaphoreType.DMA,
        pltpu.SemaphoreType.DMA,
    ],
)(_bags_body)


def kernel(token_index, emb_table, fc_w, fc_b):
    packed, bias16 = _project(emb_table.T, fc_w, fc_b[:, None])
    out2 = _bags(packed, token_index.T, bias16)
    return out2.T


# speedup vs baseline: 1.0223x; 1.0223x over previous
import functools

import jax
import jax.numpy as jnp
from jax import lax
from jax.experimental import pallas as pl
from jax.experimental.pallas import tpu as pltpu
from jax.experimental.pallas import tpu_sc as plsc

VOCAB = 100000
EMB = 64
NUM_CLASS = 2
B = 16384
L = 200

NC = 2
NS = 16
NW = NC * NS
BAGS_PER_TEC = B // NW
GROUP = 16
CHUNK = 128
NCHUNK = BAGS_PER_TEC // CHUNK
SUBG = CHUNK // GROUP
RSLICE = 40
NSLICE = L // RSLICE
TSLICES = NCHUNK * NSLICE
NBUF = 4
POS_UNROLL = 8
PBLK = 20480
VOCAB_PAD = 102400
PGRID = VOCAB_PAD // PBLK


def _rne_bf16_hi(x_f32):
    u = lax.bitcast_convert_type(x_f32, jnp.int32)
    lsb = lax.shift_right_logical(u, 16) & 1
    r = u + jnp.int32(0x7FFF) + lsb
    return r & jnp.int32(-65536)


def _proj_body(embt_ref, w_ref, b_ref, out_ref, bias_ref):
    pf = lax.dot_general(
        w_ref[...], embt_ref[...], (((1,), (0,)), ((), ())),
        preferred_element_type=jnp.float32)
    hi0 = _rne_bf16_hi(pf[0:1, :])
    hi1 = _rne_bf16_hi(pf[1:2, :])
    packed = hi1 | lax.shift_right_logical(hi0, 16)
    out_ref[...] = packed.reshape(PBLK)

    @pl.when(pl.program_id(0) == 0)
    def _():
        bias_ref[...] = jnp.broadcast_to(b_ref[...], (NUM_CLASS, 16))


def _project(emb_table_t, fc_w, fc_b2):
    return pl.pallas_call(
        _proj_body,
        grid=(PGRID,),
        in_specs=[
            pl.BlockSpec((EMB, PBLK), lambda i: (0, i)),
            pl.BlockSpec((NUM_CLASS, EMB), lambda i: (0, 0)),
            pl.BlockSpec((NUM_CLASS, 1), lambda i: (0, 0)),
        ],
        out_specs=(
            pl.BlockSpec((PBLK,), lambda i: (i,)),
            pl.BlockSpec((NUM_CLASS, 16), lambda i: (0, 0)),
        ),
        out_shape=(
            jax.ShapeDtypeStruct((VOCAB_PAD,), jnp.int32),
            jax.ShapeDtypeStruct((NUM_CLASS, 16), jnp.float32),
        ),
    )(emb_table_t, fc_w, fc_b2)


def _bags_body(ptab_hbm, tokt_hbm, bias_hbm, out_hbm,
               tab_sh, table_v, buf0_v, buf1_v, buf2_v, buf3_v,
               res_v, acc_v, bias_v, sem0, sem1, sem2, sem3):
    c = lax.axis_index("c")
    s = lax.axis_index("s")
    wid = s * NC + c
    col0 = wid * BAGS_PER_TEC
    sems = (sem0, sem1, sem2, sem3)
    bufs = (buf0_v, buf1_v, buf2_v, buf3_v)

    def start_slice_copy(t, bb):
        col = col0 + (t // NSLICE) * CHUNK
        r0 = (t % NSLICE) * RSLICE
        pltpu.async_copy(
            tokt_hbm.at[pl.ds(r0, RSLICE), pl.ds(col, CHUNK)],
            bufs[bb], sems[bb])

    def wait_slice_copy(bb):
        pltpu.make_async_copy(
            tokt_hbm.at[pl.ds(0, RSLICE), pl.ds(0, CHUNK)],
            bufs[bb], sems[bb]).wait()

    for bb in range(NBUF):
        start_slice_copy(bb, bb)

    @pl.when(s == 0)
    def _():
        pltpu.sync_copy(ptab_hbm, tab_sh)

    plsc.subcore_barrier()
    pltpu.sync_copy(tab_sh, table_v)
    pltpu.sync_copy(bias_hbm, bias_v)

    iota16 = lax.iota(jnp.int32, 16)
    b0 = bias_v[0, :]
    b1 = bias_v[1, :]
    inv_l = jnp.float32(1.0 / L)
    zerov = jnp.zeros((16,), jnp.float32)

    for j in range(SUBG):
        acc_v[pl.ds(j * GROUP, GROUP)] = zerov
        acc_v[pl.ds(CHUNK + j * GROUP, GROUP)] = zerov

    @pl.loop(0, TSLICES, step=NBUF)
    def _outer(t4):
        for bb in range(NBUF):
            t = t4 + bb
            wait_slice_copy(bb)
            rc = lax.rem(t, NSLICE)
            g = lax.div(t, NSLICE)

            @pl.loop(0, SUBG)
            def _groups(j, bb=bb):
                a0 = acc_v[pl.ds(j * GROUP, GROUP)]
                a1 = acc_v[pl.ds(CHUNK + j * GROUP, GROUP)]

                def pos_body(i, carry, bb=bb, j=j):
                    a0, a1, c0, c1 = carry
                    for k in range(POS_UNROLL):
                        p = i * POS_UNROLL + k
                        toks = bufs[bb][p, pl.ds(j * GROUP, GROUP)]
                        packed = plsc.load_gather(table_v, [toks])
                        lo = plsc.bitcast(packed << 16, jnp.float32)
                        hi = plsc.bitcast(packed & jnp.int32(-65536),
                                          jnp.float32)
                        if k % 2 == 0:
                            a0 = a0 + lo
                            a1 = a1 + hi
                        else:
                            c0 = c0 + lo
                            c1 = c1 + hi
                    return a0, a1, c0, c1

                a0, a1, c0, c1 = lax.fori_loop(
                    0, RSLICE // POS_UNROLL, pos_body,
                    (a0, a1, zerov, zerov))
                acc_v[pl.ds(j * GROUP, GROUP)] = a0 + c0
                acc_v[pl.ds(CHUNK + j * GROUP, GROUP)] = a1 + c1

            tt = t + NBUF

            @pl.when(tt < TSLICES)
            def _(bb=bb, tt=tt):
                start_slice_copy(tt, bb)

            @pl.when(rc == NSLICE - 1)
            def _(g=g):
                @pl.loop(0, SUBG)
                def _fin(j):
                    a0 = acc_v[pl.ds(j * GROUP, GROUP)]
                    a1 = acc_v[pl.ds(CHUNK + j * GROUP, GROUP)]
                    base = g * CHUNK + j * GROUP
                    res_v[0, pl.ds(base, GROUP)] = a0 * inv_l + b0
                    res_v[1, pl.ds(base, GROUP)] = a1 * inv_l + b1
                    acc_v[pl.ds(j * GROUP, GROUP)] = zerov
                    acc_v[pl.ds(CHUNK + j * GROUP, GROUP)] = zerov

    pltpu.sync_copy(res_v, out_hbm.at[:, pl.ds(col0, BAGS_PER_TEC)])


_bags = functools.partial(
    pl.kernel,
    out_type=jax.ShapeDtypeStruct((NUM_CLASS, B), jnp.float32),
    mesh=plsc.VectorSubcoreMesh(
        core_axis_name="c", subcore_axis_name="s",
        num_cores=NC, num_subcores=NS),
    compiler_params=pltpu.CompilerParams(needs_layout_passes=False),
    scratch_types=[
        pltpu.VMEM_SHARED((VOCAB_PAD,), jnp.int32),
        pltpu.VMEM((VOCAB_PAD,), jnp.int32),
        pltpu.VMEM((RSLICE, CHUNK), jnp.int32),
        pltpu.VMEM((RSLICE, CHUNK), jnp.int32),
        pltpu.VMEM((RSLICE, CHUNK), jnp.int32),
        pltpu.VMEM((RSLICE, CHUNK), jnp.int32),
        pltpu.VMEM((NUM_CLASS, BAGS_PER_TEC), jnp.float32),
        pltpu.VMEM((CHUNK * NUM_CLASS,), jnp.float32),
        pltpu.VMEM((NUM_CLASS, 16), jnp.float32),
        pltpu.SemaphoreType.DMA,
        pltpu.SemaphoreType.DMA,
        pltpu.SemaphoreType.DMA,
        pltpu.SemaphoreType.DMA,
    ],
)(_bags_body)


def kernel(token_index, emb_table, fc_w, fc_b):
    packed, bias16 = _project(emb_table.T, fc_w, fc_b[:, None])
    out2 = _bags(packed, token_index.T, bias16)
    return out2.T


# speedup vs baseline: 1.0358x; 1.0132x over previous
import functools

import jax
import jax.numpy as jnp
from jax import lax
from jax.experimental import pallas as pl
from jax.experimental.pallas import tpu as pltpu
from jax.experimental.pallas import tpu_sc as plsc

VOCAB = 100000
EMB = 64
NUM_CLASS = 2
B = 16384
L = 200

NC = 2
NS = 16
NW = NC * NS
BAGS_PER_TEC = B // NW
GROUP = 16
CHUNK = 128
NCHUNK = BAGS_PER_TEC // CHUNK
SUBG = CHUNK // GROUP
RSLICE = 40
NSLICE = L // RSLICE
TSLICES = NCHUNK * NSLICE
NBUF = 4
POS_UNROLL = 8
PBLK = 20480
VOCAB_PAD = 102400
PGRID = VOCAB_PAD // PBLK


def _rne_bf16_hi(x_f32):
    u = lax.bitcast_convert_type(x_f32, jnp.int32)
    lsb = lax.shift_right_logical(u, 16) & 1
    r = u + jnp.int32(0x7FFF) + lsb
    return r & jnp.int32(-65536)


def _proj_body(embt_ref, w_ref, out_ref):
    pf = lax.dot_general(
        w_ref[...], embt_ref[...], (((1,), (0,)), ((), ())),
        preferred_element_type=jnp.float32)
    hi0 = _rne_bf16_hi(pf[0:1, :])
    hi1 = _rne_bf16_hi(pf[1:2, :])
    packed = hi1 | lax.shift_right_logical(hi0, 16)
    out_ref[...] = packed.reshape(PBLK)


def _project(emb_table_t, fc_w):
    return pl.pallas_call(
        _proj_body,
        grid=(PGRID,),
        in_specs=[
            pl.BlockSpec((EMB, PBLK), lambda i: (0, i)),
            pl.BlockSpec((NUM_CLASS, EMB), lambda i: (0, 0)),
        ],
        out_specs=pl.BlockSpec((PBLK,), lambda i: (i,)),
        out_shape=jax.ShapeDtypeStruct((VOCAB_PAD,), jnp.int32),
    )(emb_table_t, fc_w)


def _bags_body(ptab_hbm, tokt_hbm, bias_hbm, out_hbm,
               tab_sh, table_v, buf0_v, buf1_v, buf2_v, buf3_v,
               res_v, acc_v, bias_v, sem0, sem1, sem2, sem3):
    c = lax.axis_index("c")
    s = lax.axis_index("s")
    wid = s * NC + c
    col0 = wid * BAGS_PER_TEC
    sems = (sem0, sem1, sem2, sem3)
    bufs = (buf0_v, buf1_v, buf2_v, buf3_v)

    def start_slice_copy(t, bb):
        col = col0 + (t // NSLICE) * CHUNK
        r0 = (t % NSLICE) * RSLICE
        pltpu.async_copy(
            tokt_hbm.at[pl.ds(r0, RSLICE), pl.ds(col, CHUNK)],
            bufs[bb], sems[bb])

    def wait_slice_copy(bb):
        pltpu.make_async_copy(
            tokt_hbm.at[pl.ds(0, RSLICE), pl.ds(0, CHUNK)],
            bufs[bb], sems[bb]).wait()

    for bb in range(NBUF):
        start_slice_copy(bb, bb)

    @pl.when(s == 0)
    def _():
        pltpu.sync_copy(ptab_hbm, tab_sh)

    plsc.subcore_barrier()
    pltpu.sync_copy(tab_sh, table_v)
    pltpu.sync_copy(bias_hbm, bias_v)

    iota16 = lax.iota(jnp.int32, 16)
    b0 = bias_v[0, :]
    b1 = bias_v[1, :]
    inv_l = jnp.float32(1.0 / L)
    zerov = jnp.zeros((16,), jnp.float32)

    for j in range(SUBG):
        acc_v[pl.ds(j * GROUP, GROUP)] = zerov
        acc_v[pl.ds(CHUNK + j * GROUP, GROUP)] = zerov

    @pl.loop(0, TSLICES, step=NBUF)
    def _outer(t4):
        for bb in range(NBUF):
            t = t4 + bb
            wait_slice_copy(bb)
            rc = lax.rem(t, NSLICE)
            g = lax.div(t, NSLICE)

            @pl.loop(0, SUBG)
            def _groups(j, bb=bb):
                a0 = acc_v[pl.ds(j * GROUP, GROUP)]
                a1 = acc_v[pl.ds(CHUNK + j * GROUP, GROUP)]

                def pos_body(i, carry, bb=bb, j=j):
                    a0, a1, c0, c1 = carry
                    for k in range(POS_UNROLL):
                        p = i * POS_UNROLL + k
                        toks = bufs[bb][p, pl.ds(j * GROUP, GROUP)]
                        packed = plsc.load_gather(table_v, [toks])
                        lo = plsc.bitcast(packed << 16, jnp.float32)
                        hi = plsc.bitcast(packed & jnp.int32(-65536),
                                          jnp.float32)
                        if k % 2 == 0:
                            a0 = a0 + lo
                            a1 = a1 + hi
                        else:
                            c0 = c0 + lo
                            c1 = c1 + hi
                    return a0, a1, c0, c1

                a0, a1, c0, c1 = lax.fori_loop(
                    0, RSLICE // POS_UNROLL, pos_body,
                    (a0, a1, zerov, zerov))
                acc_v[pl.ds(j * GROUP, GROUP)] = a0 + c0
                acc_v[pl.ds(CHUNK + j * GROUP, GROUP)] = a1 + c1

            tt = t + NBUF

            @pl.when(tt < TSLICES)
            def _(bb=bb, tt=tt):
                start_slice_copy(tt, bb)

            @pl.when(rc == NSLICE - 1)
            def _(g=g):
                @pl.loop(0, SUBG)
                def _fin(j):
                    a0 = acc_v[pl.ds(j * GROUP, GROUP)]
                    a1 = acc_v[pl.ds(CHUNK + j * GROUP, GROUP)]
                    base = g * CHUNK + j * GROUP
                    res_v[0, pl.ds(base, GROUP)] = a0 * inv_l + b0
                    res_v[1, pl.ds(base, GROUP)] = a1 * inv_l + b1
                    acc_v[pl.ds(j * GROUP, GROUP)] = zerov
                    acc_v[pl.ds(CHUNK + j * GROUP, GROUP)] = zerov

    pltpu.sync_copy(res_v, out_hbm.at[:, pl.ds(col0, BAGS_PER_TEC)])


_bags = functools.partial(
    pl.kernel,
    out_type=jax.ShapeDtypeStruct((NUM_CLASS, B), jnp.float32),
    mesh=plsc.VectorSubcoreMesh(
        core_axis_name="c", subcore_axis_name="s",
        num_cores=NC, num_subcores=NS),
    compiler_params=pltpu.CompilerParams(needs_layout_passes=False),
    scratch_types=[
        pltpu.VMEM_SHARED((VOCAB_PAD,), jnp.int32),
        pltpu.VMEM((VOCAB_PAD,), jnp.int32),
        pltpu.VMEM((RSLICE, CHUNK), jnp.int32),
        pltpu.VMEM((RSLICE, CHUNK), jnp.int32),
        pltpu.VMEM((RSLICE, CHUNK), jnp.int32),
        pltpu.VMEM((RSLICE, CHUNK), jnp.int32),
        pltpu.VMEM((NUM_CLASS, BAGS_PER_TEC), jnp.float32),
        pltpu.VMEM((CHUNK * NUM_CLASS,), jnp.float32),
        pltpu.VMEM((NUM_CLASS, 16), jnp.float32),
        pltpu.SemaphoreType.DMA,
        pltpu.SemaphoreType.DMA,
        pltpu.SemaphoreType.DMA,
        pltpu.SemaphoreType.DMA,
    ],
)(_bags_body)


def kernel(token_index, emb_table, fc_w, fc_b):
    packed = _project(emb_table.T, fc_w)
    bias16 = jnp.broadcast_to(fc_b[:, None], (NUM_CLASS, 16))
    out2 = _bags(packed, token_index.T, bias16)
    return out2.T
